# baseline (speedup 1.0000x reference)
import functools

import jax
import jax.numpy as jnp
from jax import lax
from jax.experimental import pallas as pl
from jax.experimental.pallas import tpu as pltpu
from jax.experimental.pallas import tpu_sc as plsc

N = 10000
E = 160000
IN_DIM = 128
OUT_DIM = 64

NC = 2
NS = 16
NW = NC * NS
CHUNK = 128
CPT = 40
E_PAD = NW * CPT * CHUNK
N_PAD = 10240
ROWS_PER_TILE = N_PAD // NS
STAGE = 320
NSLOT = CPT


def _worker_id():
    return lax.axis_index("s") * NC + lax.axis_index("c")


def _deg_body(srcm, dstm, zeros_hbm, out_hbm, acc_o, acc_i, stage,
              idx_s, idx_d, ones_v, semi, sem):
    cid = lax.axis_index("c")
    sid = lax.axis_index("s")
    wid = _worker_id()
    row0 = sid * ROWS_PER_TILE

    cp_s = pltpu.async_copy(srcm.at[pl.ds(wid * CPT, CPT)], idx_s, semi)
    cp_d = pltpu.async_copy(dstm.at[pl.ds(wid * CPT, CPT)], idx_d, semi)
    for j in range(CHUNK // 16):
        ones_v[pl.ds(j * 16, 16)] = jnp.ones((16,), jnp.float32)

    pltpu.sync_copy(zeros_hbm, stage)
    pltpu.sync_copy(stage, acc_o.at[pl.ds(row0, ROWS_PER_TILE)])
    pltpu.sync_copy(stage, acc_i.at[pl.ds(row0, ROWS_PER_TILE)])
    plsc.subcore_barrier()
    cp_s.wait()
    cp_d.wait()

    def fire(c, carry):
        pltpu.async_copy(ones_v, acc_o.at[idx_s.at[c]], sem, add=True)
        pltpu.async_copy(ones_v, acc_i.at[idx_d.at[c]], sem, add=True)
        return carry

    lax.fori_loop(0, CPT, fire, 0)

    def drain(c, carry):
        pltpu.make_async_copy(ones_v, acc_o.at[idx_s.at[c]], sem).wait()
        pltpu.make_async_copy(ones_v, acc_i.at[idx_d.at[c]], sem).wait()
        return carry

    lax.fori_loop(0, CPT, drain, 0)
    plsc.subcore_barrier()

    pltpu.sync_copy(acc_o.at[pl.ds(row0, ROWS_PER_TILE)], stage)
    pltpu.sync_copy(stage, out_hbm.at[cid, 0, pl.ds(row0, ROWS_PER_TILE)])
    pltpu.sync_copy(acc_i.at[pl.ds(row0, ROWS_PER_TILE)], stage)
    pltpu.sync_copy(stage, out_hbm.at[cid, 1, pl.ds(row0, ROWS_PER_TILE)])


@functools.cache
def _sc_degrees_kernel():
    return pl.kernel(
        _deg_body,
        out_type=jax.ShapeDtypeStruct((NC, 2, N_PAD), jnp.float32),
        mesh=plsc.VectorSubcoreMesh(core_axis_name="c", subcore_axis_name="s",
                                    num_cores=NC, num_subcores=NS),
        scratch_types=[
            pltpu.VMEM_SHARED((N_PAD,), jnp.float32),
            pltpu.VMEM_SHARED((N_PAD,), jnp.float32),
            pltpu.VMEM((ROWS_PER_TILE,), jnp.float32),
            pltpu.VMEM((CPT, CHUNK), jnp.int32),
            pltpu.VMEM((CPT, CHUNK), jnp.int32),
            pltpu.VMEM((CHUNK,), jnp.float32),
            pltpu.SemaphoreType.DMA,
            pltpu.SemaphoreType.DMA,
        ],
        compiler_params=pltpu.CompilerParams(use_tc_tiling_on_sc=False),
    )


def _segsum_body(x_hbm, srcm, dstm, zeros_hbm, out_hbm, acc, stage,
                 idx_s, idx_d, rows, semi,
                 semg0, semg1, semg2, semg3, sems0, sems1, sems2, sems3):
    cid = lax.axis_index("c")
    sid = lax.axis_index("s")
    wid = _worker_id()
    row0 = sid * ROWS_PER_TILE
    semg = (semg0, semg1, semg2, semg3)
    sems = (sems0, sems1, sems2, sems3)

    cp_s = pltpu.async_copy(srcm.at[pl.ds(wid * CPT, CPT)], idx_s, semi)
    cp_d = pltpu.async_copy(dstm.at[pl.ds(wid * CPT, CPT)], idx_d, semi)

    pltpu.sync_copy(zeros_hbm, stage)
    pltpu.sync_copy(stage, acc.at[pl.ds(row0, STAGE)])
    pltpu.sync_copy(stage, acc.at[pl.ds(row0 + STAGE, STAGE)])
    plsc.subcore_barrier()
    cp_s.wait()
    cp_d.wait()

    def gathers(s, j):
        pltpu.async_copy(x_hbm.at[idx_s.at[s]], rows.at[j], semg[j])

    def wait_gathers(s, j):
        pltpu.make_async_copy(x_hbm.at[idx_s.at[s]], rows.at[j],
                              semg[j]).wait()

    def scatters(s, j):
        pltpu.async_copy(rows.at[j], acc.at[idx_d.at[s]], sems[j], add=True)

    def wait_scatters(s, j):
        pltpu.make_async_copy(rows.at[j], acc.at[idx_d.at[s]],
                              sems[j]).wait()

    gathers(0, 0)
    gathers(1, 1)

    def body(t, carry):
        for j in range(4):
            s = t * 4 + j
            wait_gathers(s, j)
            scatters(s, j)
            j2 = (j + 2) % 4
            s2 = s + 2

            @pl.when(s2 < NSLOT)
            def _():
                @pl.when(s >= 2)
                def _():
                    wait_scatters(s - 2, j2)
                gathers(s2, j2)
        return carry

    lax.fori_loop(0, NSLOT // 4, body, 0)

    for j in range(4):
        wait_scatters(NSLOT - 4 + j, j)
    plsc.subcore_barrier()

    for k in range(ROWS_PER_TILE // STAGE):
        pltpu.sync_copy(acc.at[pl.ds(row0 + k * STAGE, STAGE)], stage)
        pltpu.sync_copy(stage, out_hbm.at[cid, pl.ds(row0 + k * STAGE, STAGE)])


@functools.cache
def _sc_segsum_kernel():
    return pl.kernel(
        _segsum_body,
        out_type=jax.ShapeDtypeStruct((NC, N_PAD, OUT_DIM), jnp.float32),
        mesh=plsc.VectorSubcoreMesh(core_axis_name="c", subcore_axis_name="s",
                                    num_cores=NC, num_subcores=NS),
        scratch_types=[
            pltpu.VMEM_SHARED((N_PAD, OUT_DIM), jnp.float32),
            pltpu.VMEM((STAGE, OUT_DIM), jnp.float32),
            pltpu.VMEM((CPT, CHUNK), jnp.int32),
            pltpu.VMEM((CPT, CHUNK), jnp.int32),
            pltpu.VMEM((4, CHUNK, OUT_DIM), jnp.float32),
            pltpu.SemaphoreType.DMA,
            pltpu.SemaphoreType.DMA,
            pltpu.SemaphoreType.DMA,
            pltpu.SemaphoreType.DMA,
            pltpu.SemaphoreType.DMA,
            pltpu.SemaphoreType.DMA,
            pltpu.SemaphoreType.DMA,
            pltpu.SemaphoreType.DMA,
            pltpu.SemaphoreType.DMA,
        ],
        compiler_params=pltpu.CompilerParams(use_tc_tiling_on_sc=False),
    )


def _norms(d):
    od = d[0, 0] + d[1, 0]
    idg = d[0, 1] + d[1, 1]
    ns = jnp.where(od > 0, lax.rsqrt(jnp.maximum(od, 1.0)), 0.0)
    nd = jnp.where(idg > 0, lax.rsqrt(jnp.maximum(idg, 1.0)), 0.0)
    return ns, nd


def _mm1_body(h_ref, w1_ref, u_ref):
    u_ref[...] = jnp.dot(h_ref[...], w1_ref[...],
                         preferred_element_type=jnp.float32)


_tc_mm1 = pl.pallas_call(
    _mm1_body,
    out_shape=jax.ShapeDtypeStruct((N, OUT_DIM), jnp.float32),
)


def _scale_body(u_ref, deg_ref, x1_ref):
    ns, _ = _norms(deg_ref[...])
    x1_ref[:N] = u_ref[...] * ns[:N]
    x1_ref[N:] = jnp.zeros((N_PAD - N, OUT_DIM), jnp.float32)


_tc_scale = pl.pallas_call(
    _scale_body,
    out_shape=jax.ShapeDtypeStruct((N_PAD, OUT_DIM), jnp.float32),
)


def _z_body(p_ref, deg_ref, b1_ref, z_ref, zn_ref):
    ns, nd = _norms(deg_ref[...])
    p = p_ref[...]
    s = (p[0, :N] + p[1, :N]) * nd[:N] + b1_ref[...]
    z = jax.nn.sigmoid(s)
    z_ref[...] = z
    zn_ref[:N] = z * ns[:N]
    zn_ref[N:] = jnp.zeros((N_PAD - N, OUT_DIM), jnp.float32)


_tc_z = pl.pallas_call(
    _z_body,
    out_shape=(
        jax.ShapeDtypeStruct((N, OUT_DIM), jnp.float32),
        jax.ShapeDtypeStruct((N_PAD, OUT_DIM), jnp.float32),
    ),
)


def _xhat_body(q_ref, deg_ref, w2_ref, b2_ref, xh_ref):
    _, nd = _norms(deg_ref[...])
    q = q_ref[...]
    a = (q[0, :N] + q[1, :N]) * nd[:N]
    s = jnp.dot(a, w2_ref[...], preferred_element_type=jnp.float32) + b2_ref[...]
    xh_ref[...] = jax.nn.sigmoid(s)


_tc_xhat = pl.pallas_call(
    _xhat_body,
    out_shape=jax.ShapeDtypeStruct((N, IN_DIM), jnp.float32),
)


BR = 512
BC = 2048
_GR = pl.cdiv(N, BR)
_GC = pl.cdiv(N, BC)


def _struct_body(zr_ref, zc_ref, out_ref):
    s = lax.dot_general(zr_ref[...], zc_ref[...],
                        (((1,), (1,)), ((), ())),
                        preferred_element_type=jnp.float32)
    out_ref[...] = jax.nn.sigmoid(s)


_tc_struct = pl.pallas_call(
    _struct_body,
    grid=(_GC, _GR),
    in_specs=[
        pl.BlockSpec((BR, OUT_DIM), lambda j, i: (i, 0)),
        pl.BlockSpec((BC, OUT_DIM), lambda j, i: (j, 0)),
    ],
    out_specs=pl.BlockSpec((BR, BC), lambda j, i: (i, j)),
    out_shape=jax.ShapeDtypeStruct((N, N), jnp.float32),
)


def kernel(h, edge_index, W1, b1, W2, b2):
    pad = jnp.full((E_PAD - E,), N, jnp.int32)
    src2d = jnp.concatenate([edge_index[0], pad]).reshape(E_PAD // CHUNK, CHUNK)
    dst2d = jnp.concatenate([edge_index[1], pad]).reshape(E_PAD // CHUNK, CHUNK)
    zeros_1d = jnp.zeros((ROWS_PER_TILE,), jnp.float32)
    zeros_row = jnp.zeros((STAGE, OUT_DIM), jnp.float32)

    degs = _sc_degrees_kernel()(src2d, dst2d, zeros_1d)
    degs4 = degs.reshape(NC, 2, N_PAD, 1)

    u = _tc_mm1(h, W1)
    x1 = _tc_scale(u, degs4)
    p = _sc_segsum_kernel()(x1, src2d, dst2d, zeros_row)
    z, zn = _tc_z(p, degs4, b1[None, :])
    q = _sc_segsum_kernel()(zn, src2d, dst2d, zeros_row)
    x_hat = _tc_xhat(q, degs4, W2, b2[None, :])
    struct = _tc_struct(z, z)
    return (struct, x_hat)


# speedup vs baseline: 1.1187x; 1.1187x over previous
import functools

import jax
import jax.numpy as jnp
from jax import lax
from jax.experimental import pallas as pl
from jax.experimental.pallas import tpu as pltpu
from jax.experimental.pallas import tpu_sc as plsc

N = 10000
E = 160000
IN_DIM = 128
OUT_DIM = 64

NC = 2
NS = 16
CHUNK = 128
CPT0 = 56
CPT1 = 24
NCHUNKS = NS * (CPT0 + CPT1)
EPC = NS * CPT0 + NS * CPT1 + (CPT0 - CPT1)
EPC = 1312
N_PAD = 10240
ROWS_PER_TILE = N_PAD // NS
STAGE = 320


def _tile_range():
    cid = lax.axis_index("c")
    sid = lax.axis_index("s")
    nch = jnp.where(cid == 0, CPT0, CPT1)
    base = jnp.where(cid == 0, sid * CPT0, NS * CPT0 + sid * CPT1)
    return cid, sid, base, nch


def _deg_body(srcm, dstm, zeros_hbm, out_hbm, acc_o, acc_i, stage,
              idx_s, idx_d, ones_v, semi, sem):
    cid, sid, base, nch = _tile_range()
    row0 = sid * ROWS_PER_TILE

    cp_s = pltpu.async_copy(srcm.at[pl.ds(base, CPT0)], idx_s, semi)
    cp_d = pltpu.async_copy(dstm.at[pl.ds(base, CPT0)], idx_d, semi)
    for j in range(CHUNK // 16):
        ones_v[pl.ds(j * 16, 16)] = jnp.ones((16,), jnp.float32)

    pltpu.sync_copy(zeros_hbm, stage)
    pltpu.sync_copy(stage, acc_o.at[pl.ds(row0, ROWS_PER_TILE)])
    pltpu.sync_copy(stage, acc_i.at[pl.ds(row0, ROWS_PER_TILE)])
    plsc.subcore_barrier()
    cp_s.wait()
    cp_d.wait()

    def fire(c, carry):
        pltpu.async_copy(ones_v, acc_o.at[idx_s.at[c]], sem, add=True)
        pltpu.async_copy(ones_v, acc_i.at[idx_d.at[c]], sem, add=True)
        return carry

    lax.fori_loop(0, nch, fire, 0)

    def drain(c, carry):
        pltpu.make_async_copy(ones_v, acc_o.at[idx_s.at[c]], sem).wait()
        pltpu.make_async_copy(ones_v, acc_i.at[idx_d.at[c]], sem).wait()
        return carry

    lax.fori_loop(0, nch, drain, 0)
    plsc.subcore_barrier()

    pltpu.sync_copy(acc_o.at[pl.ds(row0, ROWS_PER_TILE)], stage)
    pltpu.sync_copy(stage, out_hbm.at[cid, 0, pl.ds(row0, ROWS_PER_TILE)])
    pltpu.sync_copy(acc_i.at[pl.ds(row0, ROWS_PER_TILE)], stage)
    pltpu.sync_copy(stage, out_hbm.at[cid, 1, pl.ds(row0, ROWS_PER_TILE)])


@functools.cache
def _sc_degrees_kernel():
    return pl.kernel(
        _deg_body,
        out_type=jax.ShapeDtypeStruct((NC, 2, N_PAD), jnp.float32),
        mesh=plsc.VectorSubcoreMesh(core_axis_name="c", subcore_axis_name="s",
                                    num_cores=NC, num_subcores=NS),
        scratch_types=[
            pltpu.VMEM_SHARED((N_PAD,), jnp.float32),
            pltpu.VMEM_SHARED((N_PAD,), jnp.float32),
            pltpu.VMEM((ROWS_PER_TILE,), jnp.float32),
            pltpu.VMEM((CPT0, CHUNK), jnp.int32),
            pltpu.VMEM((CPT0, CHUNK), jnp.int32),
            pltpu.VMEM((CHUNK,), jnp.float32),
            pltpu.SemaphoreType.DMA,
            pltpu.SemaphoreType.DMA,
        ],
        compiler_params=pltpu.CompilerParams(use_tc_tiling_on_sc=False),
    )


def _segsum_body(x_hbm, srcm, dstm, zeros_hbm, out_hbm, acc, stage,
                 idx_s, idx_d, rows, semi,
                 semg0, semg1, semg2, semg3, sems0, sems1, sems2, sems3):
    cid, sid, base, nch = _tile_range()
    row0 = sid * ROWS_PER_TILE
    semg = (semg0, semg1, semg2, semg3)
    sems = (sems0, sems1, sems2, sems3)

    cp_s = pltpu.async_copy(srcm.at[pl.ds(base, CPT0)], idx_s, semi)
    cp_d = pltpu.async_copy(dstm.at[pl.ds(base, CPT0)], idx_d, semi)

    pltpu.sync_copy(zeros_hbm, stage)
    pltpu.sync_copy(stage, acc.at[pl.ds(row0, STAGE)])
    pltpu.sync_copy(stage, acc.at[pl.ds(row0 + STAGE, STAGE)])
    plsc.subcore_barrier()
    cp_s.wait()
    cp_d.wait()

    def gathers(s, j):
        pltpu.async_copy(x_hbm.at[idx_s.at[s]], rows.at[j], semg[j])

    def wait_gathers(s, j):
        pltpu.make_async_copy(x_hbm.at[idx_s.at[s]], rows.at[j],
                              semg[j]).wait()

    def scatters(s, j):
        pltpu.async_copy(rows.at[j], acc.at[idx_d.at[s]], sems[j], add=True)

    def wait_scatters(s, j):
        pltpu.make_async_copy(rows.at[j], acc.at[idx_d.at[s]],
                              sems[j]).wait()

    gathers(0, 0)
    gathers(1, 1)

    def body(t, carry):
        for j in range(4):
            s = t * 4 + j
            wait_gathers(s, j)
            scatters(s, j)
            j2 = (j + 2) % 4
            s2 = s + 2

            @pl.when(s2 < nch)
            def _():
                @pl.when(s >= 2)
                def _():
                    wait_scatters(s - 2, j2)
                gathers(s2, j2)
        return carry

    lax.fori_loop(0, nch // 4, body, 0)

    for j in range(4):
        wait_scatters(nch - 4 + j, j)
    plsc.subcore_barrier()

    for k in range(ROWS_PER_TILE // STAGE):
        pltpu.sync_copy(acc.at[pl.ds(row0 + k * STAGE, STAGE)], stage)
        pltpu.sync_copy(stage, out_hbm.at[cid, pl.ds(row0 + k * STAGE, STAGE)])


@functools.cache
def _sc_segsum_kernel():
    return pl.kernel(
        _segsum_body,
        out_type=jax.ShapeDtypeStruct((NC, N_PAD, OUT_DIM), jnp.float32),
        mesh=plsc.VectorSubcoreMesh(core_axis_name="c", subcore_axis_name="s",
                                    num_cores=NC, num_subcores=NS),
        scratch_types=[
            pltpu.VMEM_SHARED((N_PAD, OUT_DIM), jnp.float32),
            pltpu.VMEM((STAGE, OUT_DIM), jnp.float32),
            pltpu.VMEM((CPT0, CHUNK), jnp.int32),
            pltpu.VMEM((CPT0, CHUNK), jnp.int32),
            pltpu.VMEM((4, CHUNK, OUT_DIM), jnp.float32),
            pltpu.SemaphoreType.DMA,
            pltpu.SemaphoreType.DMA,
            pltpu.SemaphoreType.DMA,
            pltpu.SemaphoreType.DMA,
            pltpu.SemaphoreType.DMA,
            pltpu.SemaphoreType.DMA,
            pltpu.SemaphoreType.DMA,
            pltpu.SemaphoreType.DMA,
            pltpu.SemaphoreType.DMA,
        ],
        compiler_params=pltpu.CompilerParams(use_tc_tiling_on_sc=False),
    )


def _norms(d):
    od = d[0, 0] + d[1, 0]
    idg = d[0, 1] + d[1, 1]
    ns = jnp.where(od > 0, lax.rsqrt(jnp.maximum(od, 1.0)), 0.0)
    nd = jnp.where(idg > 0, lax.rsqrt(jnp.maximum(idg, 1.0)), 0.0)
    return ns[:N].reshape(N, 1), nd[:N].reshape(N, 1)


def _mm1_body(h_ref, w1_ref, u_ref):
    u_ref[...] = jnp.dot(h_ref[...], w1_ref[...],
                         preferred_element_type=jnp.float32)


_tc_mm1 = pl.pallas_call(
    _mm1_body,
    out_shape=jax.ShapeDtypeStruct((N, OUT_DIM), jnp.float32),
)


def _scale_body(u_ref, deg_ref, x1_ref):
    ns, _ = _norms(deg_ref[...])
    x1_ref[:N] = u_ref[...] * ns
    x1_ref[N:] = jnp.zeros((N_PAD - N, OUT_DIM), jnp.float32)


_tc_scale = pl.pallas_call(
    _scale_body,
    out_shape=jax.ShapeDtypeStruct((N_PAD, OUT_DIM), jnp.float32),
)


def _z_body(p_ref, deg_ref, b1_ref, z_ref, zn_ref):
    ns, nd = _norms(deg_ref[...])
    p = p_ref[...]
    s = (p[0, :N] + p[1, :N]) * nd + b1_ref[...]
    z = jax.nn.sigmoid(s)
    z_ref[...] = z
    zn_ref[:N] = z * ns
    zn_ref[N:] = jnp.zeros((N_PAD - N, OUT_DIM), jnp.float32)


_tc_z = pl.pallas_call(
    _z_body,
    out_shape=(
        jax.ShapeDtypeStruct((N, OUT_DIM), jnp.float32),
        jax.ShapeDtypeStruct((N_PAD, OUT_DIM), jnp.float32),
    ),
)


def _xhat_body(q_ref, deg_ref, w2_ref, b2_ref, xh_ref):
    _, nd = _norms(deg_ref[...])
    q = q_ref[...]
    a = (q[0, :N] + q[1, :N]) * nd
    s = jnp.dot(a, w2_ref[...], preferred_element_type=jnp.float32) + b2_ref[...]
    xh_ref[...] = jax.nn.sigmoid(s)


_tc_xhat = pl.pallas_call(
    _xhat_body,
    out_shape=jax.ShapeDtypeStruct((N, IN_DIM), jnp.float32),
)


BR = 512
BC = 2048
_GR = pl.cdiv(N, BR)
_GC = pl.cdiv(N, BC)


def _struct_body(zr_ref, zc_ref, out_ref):
    s = lax.dot_general(zr_ref[...], zc_ref[...],
                        (((1,), (1,)), ((), ())),
                        preferred_element_type=jnp.float32)
    out_ref[...] = jax.nn.sigmoid(s)


_tc_struct = pl.pallas_call(
    _struct_body,
    grid=(_GC, _GR),
    in_specs=[
        pl.BlockSpec((BR, OUT_DIM), lambda j, i: (i, 0)),
        pl.BlockSpec((BC, OUT_DIM), lambda j, i: (j, 0)),
    ],
    out_specs=pl.BlockSpec((BR, BC), lambda j, i: (i, j)),
    out_shape=jax.ShapeDtypeStruct((N, N), jnp.float32),
)


def kernel(h, edge_index, W1, b1, W2, b2):
    pad = jnp.full((EPC * CHUNK - E,), N, jnp.int32)
    src2d = jnp.concatenate([edge_index[0], pad]).reshape(EPC, CHUNK)
    dst2d = jnp.concatenate([edge_index[1], pad]).reshape(EPC, CHUNK)
    zeros_1d = jnp.zeros((ROWS_PER_TILE,), jnp.float32)
    zeros_row = jnp.zeros((STAGE, OUT_DIM), jnp.float32)

    degs = _sc_degrees_kernel()(src2d, dst2d, zeros_1d)

    u = _tc_mm1(h, W1)
    x1 = _tc_scale(u, degs)
    p = _sc_segsum_kernel()(x1, src2d, dst2d, zeros_row)
    z, zn = _tc_z(p, degs, b1[None, :])
    q = _sc_segsum_kernel()(zn, src2d, dst2d, zeros_row)
    x_hat = _tc_xhat(q, degs, W2, b2[None, :])
    struct = _tc_struct(z, z)
    return (struct, x_hat)


# speedup vs baseline: 1.6088x; 1.4381x over previous
import functools

import jax
import jax.numpy as jnp
from jax import lax
from jax.experimental import pallas as pl
from jax.experimental.pallas import tpu as pltpu
from jax.experimental.pallas import tpu_sc as plsc

N = 10000
E = 160000
IN_DIM = 128
OUT_DIM = 64

NC = 2
NS = 16
CHUNK = 128
CPT0 = 40
CPT1 = 40
NCHUNKS = NS * (CPT0 + CPT1)
EPC = NCHUNKS + (CPT0 - CPT1)
N_PAD = 10240
ROWS_PER_TILE = N_PAD // NS
STAGE = 320


def _tile_range():
    cid = lax.axis_index("c")
    sid = lax.axis_index("s")
    nch = jnp.where(cid == 0, CPT0, CPT1)
    base = jnp.where(cid == 0, sid * CPT0, NS * CPT0 + sid * CPT1)
    return cid, sid, base, nch


def _deg_body(srcm, dstm, zeros_hbm, out_hbm, acc_o, acc_i, stage,
              idx_s, idx_d, ones_v, semi, sem):
    cid, sid, base, nch = _tile_range()
    row0 = sid * ROWS_PER_TILE

    cp_s = pltpu.async_copy(srcm.at[pl.ds(base, CPT0)], idx_s, semi)
    cp_d = pltpu.async_copy(dstm.at[pl.ds(base, CPT0)], idx_d, semi)
    for j in range(CHUNK // 16):
        ones_v[pl.ds(j * 16, 16)] = jnp.ones((16,), jnp.float32)

    pltpu.sync_copy(zeros_hbm, stage)
    pltpu.sync_copy(stage, acc_o.at[pl.ds(row0, ROWS_PER_TILE)])
    pltpu.sync_copy(stage, acc_i.at[pl.ds(row0, ROWS_PER_TILE)])
    plsc.subcore_barrier()
    cp_s.wait()
    cp_d.wait()

    def fire(c, carry):
        pltpu.async_copy(ones_v, acc_o.at[idx_s.at[c]], sem, add=True)
        pltpu.async_copy(ones_v, acc_i.at[idx_d.at[c]], sem, add=True)
        return carry

    lax.fori_loop(0, nch, fire, 0)

    def drain(c, carry):
        pltpu.make_async_copy(ones_v, acc_o.at[idx_s.at[c]], sem).wait()
        pltpu.make_async_copy(ones_v, acc_i.at[idx_d.at[c]], sem).wait()
        return carry

    lax.fori_loop(0, nch, drain, 0)
    plsc.subcore_barrier()

    pltpu.sync_copy(acc_o.at[pl.ds(row0, ROWS_PER_TILE)], stage)
    pltpu.sync_copy(stage, out_hbm.at[cid, 0, pl.ds(row0, ROWS_PER_TILE)])
    pltpu.sync_copy(acc_i.at[pl.ds(row0, ROWS_PER_TILE)], stage)
    pltpu.sync_copy(stage, out_hbm.at[cid, 1, pl.ds(row0, ROWS_PER_TILE)])


@functools.cache
def _sc_degrees_kernel():
    return pl.kernel(
        _deg_body,
        out_type=jax.ShapeDtypeStruct((NC, 2, N_PAD), jnp.float32),
        mesh=plsc.VectorSubcoreMesh(core_axis_name="c", subcore_axis_name="s",
                                    num_cores=NC, num_subcores=NS),
        scratch_types=[
            pltpu.VMEM_SHARED((N_PAD,), jnp.float32),
            pltpu.VMEM_SHARED((N_PAD,), jnp.float32),
            pltpu.VMEM((ROWS_PER_TILE,), jnp.float32),
            pltpu.VMEM((CPT0, CHUNK), jnp.int32),
            pltpu.VMEM((CPT0, CHUNK), jnp.int32),
            pltpu.VMEM((CHUNK,), jnp.float32),
            pltpu.SemaphoreType.DMA,
            pltpu.SemaphoreType.DMA,
        ],
        compiler_params=pltpu.CompilerParams(use_tc_tiling_on_sc=False),
    )


def _segsum_body(x_hbm, srcm, dstm, zeros_hbm, out_hbm, acc, xsp,
                 idx_s, idx_d, rows, semi,
                 semg0, semg1, semg2, semg3, sems0, sems1, sems2, sems3):
    cid, sid, base, nch = _tile_range()
    row0 = sid * ROWS_PER_TILE
    semg = (semg0, semg1, semg2, semg3)
    sems = (sems0, sems1, sems2, sems3)

    cp_s = pltpu.async_copy(srcm.at[pl.ds(base, CPT0)], idx_s, semi)
    cp_d = pltpu.async_copy(dstm.at[pl.ds(base, CPT0)], idx_d, semi)

    pltpu.sync_copy(x_hbm.at[pl.ds(row0, ROWS_PER_TILE)],
                    xsp.at[pl.ds(row0, ROWS_PER_TILE)])
    pltpu.sync_copy(zeros_hbm, acc.at[pl.ds(row0, ROWS_PER_TILE)])
    plsc.subcore_barrier()
    cp_s.wait()
    cp_d.wait()

    def gathers(s, j):
        pltpu.async_copy(xsp.at[idx_s.at[s]], rows.at[j], semg[j])

    def wait_gathers(s, j):
        pltpu.make_async_copy(xsp.at[idx_s.at[s]], rows.at[j],
                              semg[j]).wait()

    def scatters(s, j):
        pltpu.async_copy(rows.at[j], acc.at[idx_d.at[s]], sems[j], add=True)

    def wait_scatters(s, j):
        pltpu.make_async_copy(rows.at[j], acc.at[idx_d.at[s]],
                              sems[j]).wait()

    gathers(0, 0)
    gathers(1, 1)

    def body(t, carry):
        for j in range(4):
            s = t * 4 + j
            wait_gathers(s, j)
            scatters(s, j)
            j2 = (j + 2) % 4
            s2 = s + 2

            @pl.when(s2 < nch)
            def _():
                @pl.when(s >= 2)
                def _():
                    wait_scatters(s - 2, j2)
                gathers(s2, j2)
        return carry

    lax.fori_loop(0, nch // 4, body, 0)

    for j in range(4):
        wait_scatters(nch - 4 + j, j)
    plsc.subcore_barrier()

    pltpu.sync_copy(acc.at[pl.ds(row0, ROWS_PER_TILE)],
                    out_hbm.at[cid, pl.ds(row0, ROWS_PER_TILE)])


@functools.cache
def _sc_segsum_kernel():
    return pl.kernel(
        _segsum_body,
        out_type=jax.ShapeDtypeStruct((NC, N_PAD, OUT_DIM), jnp.float32),
        mesh=plsc.VectorSubcoreMesh(core_axis_name="c", subcore_axis_name="s",
                                    num_cores=NC, num_subcores=NS),
        scratch_types=[
            pltpu.VMEM_SHARED((N_PAD, OUT_DIM), jnp.float32),
            pltpu.VMEM_SHARED((N_PAD, OUT_DIM), jnp.float32),
            pltpu.VMEM((CPT0, CHUNK), jnp.int32),
            pltpu.VMEM((CPT0, CHUNK), jnp.int32),
            pltpu.VMEM((4, CHUNK, OUT_DIM), jnp.float32),
            pltpu.SemaphoreType.DMA,
            pltpu.SemaphoreType.DMA,
            pltpu.SemaphoreType.DMA,
            pltpu.SemaphoreType.DMA,
            pltpu.SemaphoreType.DMA,
            pltpu.SemaphoreType.DMA,
            pltpu.SemaphoreType.DMA,
            pltpu.SemaphoreType.DMA,
            pltpu.SemaphoreType.DMA,
        ],
        compiler_params=pltpu.CompilerParams(use_tc_tiling_on_sc=False),
    )


def _norms(d):
    od = d[0, 0] + d[1, 0]
    idg = d[0, 1] + d[1, 1]
    ns = jnp.where(od > 0, lax.rsqrt(jnp.maximum(od, 1.0)), 0.0)
    nd = jnp.where(idg > 0, lax.rsqrt(jnp.maximum(idg, 1.0)), 0.0)
    return ns[:N].reshape(N, 1), nd[:N].reshape(N, 1)


def _mm1_body(h_ref, w1_ref, u_ref):
    u_ref[...] = jnp.dot(h_ref[...], w1_ref[...],
                         preferred_element_type=jnp.float32)


_tc_mm1 = pl.pallas_call(
    _mm1_body,
    out_shape=jax.ShapeDtypeStruct((N, OUT_DIM), jnp.float32),
)


def _scale_body(u_ref, deg_ref, x1_ref):
    ns, _ = _norms(deg_ref[...])
    x1_ref[:N] = u_ref[...] * ns
    x1_ref[N:] = jnp.zeros((N_PAD - N, OUT_DIM), jnp.float32)


_tc_scale = pl.pallas_call(
    _scale_body,
    out_shape=jax.ShapeDtypeStruct((N_PAD, OUT_DIM), jnp.float32),
)


def _z_body(p_ref, deg_ref, b1_ref, z_ref, zn_ref):
    ns, nd = _norms(deg_ref[...])
    p = p_ref[...]
    s = (p[0, :N] + p[1, :N]) * nd + b1_ref[...]
    z = jax.nn.sigmoid(s)
    z_ref[...] = z
    zn_ref[:N] = z * ns
    zn_ref[N:] = jnp.zeros((N_PAD - N, OUT_DIM), jnp.float32)


_tc_z = pl.pallas_call(
    _z_body,
    out_shape=(
        jax.ShapeDtypeStruct((N, OUT_DIM), jnp.float32),
        jax.ShapeDtypeStruct((N_PAD, OUT_DIM), jnp.float32),
    ),
)


def _xhat_body(q_ref, deg_ref, w2_ref, b2_ref, xh_ref):
    _, nd = _norms(deg_ref[...])
    q = q_ref[...]
    a = (q[0, :N] + q[1, :N]) * nd
    s = jnp.dot(a, w2_ref[...], preferred_element_type=jnp.float32) + b2_ref[...]
    xh_ref[...] = jax.nn.sigmoid(s)


_tc_xhat = pl.pallas_call(
    _xhat_body,
    out_shape=jax.ShapeDtypeStruct((N, IN_DIM), jnp.float32),
)


BR = 512
BC = 2048
_GR = pl.cdiv(N, BR)
_GC = pl.cdiv(N, BC)


def _struct_body(zr_ref, zc_ref, out_ref):
    s = lax.dot_general(zr_ref[...], zc_ref[...],
                        (((1,), (1,)), ((), ())),
                        preferred_element_type=jnp.float32)
    out_ref[...] = jax.nn.sigmoid(s)


_tc_struct = pl.pallas_call(
    _struct_body,
    grid=(_GC, _GR),
    in_specs=[
        pl.BlockSpec((BR, OUT_DIM), lambda j, i: (i, 0)),
        pl.BlockSpec((BC, OUT_DIM), lambda j, i: (j, 0)),
    ],
    out_specs=pl.BlockSpec((BR, BC), lambda j, i: (i, j)),
    out_shape=jax.ShapeDtypeStruct((N, N), jnp.float32),
)


def kernel(h, edge_index, W1, b1, W2, b2):
    pad = jnp.full((EPC * CHUNK - E,), N, jnp.int32)
    src2d = jnp.concatenate([edge_index[0], pad]).reshape(EPC, CHUNK)
    dst2d = jnp.concatenate([edge_index[1], pad]).reshape(EPC, CHUNK)
    zeros_1d = jnp.zeros((ROWS_PER_TILE,), jnp.float32)
    zeros_row = jnp.zeros((ROWS_PER_TILE, OUT_DIM), jnp.float32)

    degs = _sc_degrees_kernel()(src2d, dst2d, zeros_1d)

    u = _tc_mm1(h, W1)
    x1 = _tc_scale(u, degs)
    p = _sc_segsum_kernel()(x1, src2d, dst2d, zeros_row)
    z, zn = _tc_z(p, degs, b1[None, :])
    q = _sc_segsum_kernel()(zn, src2d, dst2d, zeros_row)
    x_hat = _tc_xhat(q, degs, W2, b2[None, :])
    struct = _tc_struct(z, z)
    return (struct, x_hat)


# speedup vs baseline: 1.8427x; 1.1454x over previous
import functools

import jax
import jax.numpy as jnp
from jax import lax
from jax.experimental import pallas as pl
from jax.experimental.pallas import tpu as pltpu
from jax.experimental.pallas import tpu_sc as plsc

N = 10000
E = 160000
IN_DIM = 128
OUT_DIM = 64

NC = 2
NS = 16
CHUNK = 128
CPT0 = 40
CPT1 = 40
NCHUNKS = NS * (CPT0 + CPT1)
EPC = NCHUNKS + (CPT0 - CPT1)
N_PAD = 10240
ROWS_PER_TILE = N_PAD // NS
STAGE = 320


def _tile_range():
    cid = lax.axis_index("c")
    sid = lax.axis_index("s")
    nch = jnp.where(cid == 0, CPT0, CPT1)
    base = jnp.where(cid == 0, sid * CPT0, NS * CPT0 + sid * CPT1)
    return cid, sid, base, nch


def _deg_body(srcm, dstm, zeros_hbm, out_hbm, acc_o, acc_i, stage,
              idx_s, idx_d, ones_v, semi, sem):
    cid, sid, base, nch = _tile_range()
    row0 = sid * ROWS_PER_TILE

    cp_s = pltpu.async_copy(srcm.at[pl.ds(base, CPT0)], idx_s, semi)
    cp_d = pltpu.async_copy(dstm.at[pl.ds(base, CPT0)], idx_d, semi)

    for j in range(CHUNK // 16):
        ones_v[pl.ds(j * 16, 16)] = jnp.ones((16,), jnp.float32)

    pltpu.sync_copy(zeros_hbm, stage)
    pltpu.sync_copy(stage, acc_o.at[pl.ds(row0, ROWS_PER_TILE)])
    pltpu.sync_copy(stage, acc_i.at[pl.ds(row0, ROWS_PER_TILE)])
    plsc.subcore_barrier()
    cp_s.wait()
    cp_d.wait()

    def fire(c, carry):
        pltpu.async_copy(ones_v, acc_o.at[idx_s.at[c]], sem, add=True)
        pltpu.async_copy(ones_v, acc_i.at[idx_d.at[c]], sem, add=True)
        return carry

    lax.fori_loop(0, nch, fire, 0)

    def drain(c, carry):
        pltpu.make_async_copy(ones_v, acc_o.at[idx_s.at[c]], sem).wait()
        pltpu.make_async_copy(ones_v, acc_i.at[idx_d.at[c]], sem).wait()
        return carry

    lax.fori_loop(0, nch, drain, 0)
    plsc.subcore_barrier()

    pltpu.sync_copy(acc_o.at[pl.ds(row0, ROWS_PER_TILE)], stage)
    pltpu.sync_copy(stage, out_hbm.at[cid, 0, pl.ds(row0, ROWS_PER_TILE)])
    pltpu.sync_copy(acc_i.at[pl.ds(row0, ROWS_PER_TILE)], stage)
    pltpu.sync_copy(stage, out_hbm.at[cid, 1, pl.ds(row0, ROWS_PER_TILE)])


@functools.cache
def _sc_degrees_kernel():
    return pl.kernel(
        _deg_body,
        out_type=jax.ShapeDtypeStruct((NC, 2, N_PAD), jnp.float32),
        mesh=plsc.VectorSubcoreMesh(core_axis_name="c", subcore_axis_name="s",
                                    num_cores=NC, num_subcores=NS),
        scratch_types=[
            pltpu.VMEM_SHARED((N_PAD,), jnp.float32),
            pltpu.VMEM_SHARED((N_PAD,), jnp.float32),
            pltpu.VMEM((ROWS_PER_TILE,), jnp.float32),
            pltpu.VMEM((CPT0, CHUNK), jnp.int32),
            pltpu.VMEM((CPT0, CHUNK), jnp.int32),
            pltpu.VMEM((CHUNK,), jnp.float32),
            pltpu.SemaphoreType.DMA,
            pltpu.SemaphoreType.DMA,
        ],
        compiler_params=pltpu.CompilerParams(use_tc_tiling_on_sc=False),
    )


def _segsum_body(x_hbm, srcm, dstm, zeros_hbm, out_hbm, acc, xsp,
                 idx_s, idx_d, rows, semi,
                 semg0, semg1, semg2, semg3, sems0, sems1, sems2, sems3):
    cid, sid, base, nch = _tile_range()
    row0 = sid * ROWS_PER_TILE
    semg = (semg0, semg1, semg2, semg3)
    sems = (sems0, sems1, sems2, sems3)

    cp_s = pltpu.async_copy(srcm.at[pl.ds(base, CPT0)], idx_s, semi)
    cp_d = pltpu.async_copy(dstm.at[pl.ds(base, CPT0)], idx_d, semi)

    pltpu.sync_copy(x_hbm.at[pl.ds(row0, ROWS_PER_TILE)],
                    xsp.at[pl.ds(row0, ROWS_PER_TILE)])
    pltpu.sync_copy(zeros_hbm, acc.at[pl.ds(row0, ROWS_PER_TILE)])
    plsc.subcore_barrier()
    cp_s.wait()
    cp_d.wait()

    def gathers(s, j):
        pltpu.async_copy(xsp.at[idx_s.at[s]], rows.at[j], semg[j])

    def wait_gathers(s, j):
        pltpu.make_async_copy(xsp.at[idx_s.at[s]], rows.at[j],
                              semg[j]).wait()

    def scatters(s, j):
        pltpu.async_copy(rows.at[j], acc.at[idx_d.at[s]], sems[j], add=True)

    def wait_scatters(s, j):
        pltpu.make_async_copy(rows.at[j], acc.at[idx_d.at[s]],
                              sems[j]).wait()

    gathers(0, 0)
    gathers(1, 1)

    def body(t, carry):
        for j in range(4):
            s = t * 4 + j
            wait_gathers(s, j)
            scatters(s, j)
            j2 = (j + 2) % 4
            s2 = s + 2

            @pl.when(s2 < nch)
            def _():
                @pl.when(s >= 2)
                def _():
                    wait_scatters(s - 2, j2)
                gathers(s2, j2)
        return carry

    lax.fori_loop(0, nch // 4, body, 0)

    for j in range(4):
        wait_scatters(nch - 4 + j, j)
    plsc.subcore_barrier()

    pltpu.sync_copy(acc.at[pl.ds(row0, ROWS_PER_TILE)],
                    out_hbm.at[cid, pl.ds(row0, ROWS_PER_TILE)])


@functools.cache
def _sc_segsum_kernel():
    return pl.kernel(
        _segsum_body,
        out_type=jax.ShapeDtypeStruct((NC, N_PAD, OUT_DIM), jnp.float32),
        mesh=plsc.VectorSubcoreMesh(core_axis_name="c", subcore_axis_name="s",
                                    num_cores=NC, num_subcores=NS),
        scratch_types=[
            pltpu.VMEM_SHARED((N_PAD, OUT_DIM), jnp.float32),
            pltpu.VMEM_SHARED((N_PAD, OUT_DIM), jnp.float32),
            pltpu.VMEM((CPT0, CHUNK), jnp.int32),
            pltpu.VMEM((CPT0, CHUNK), jnp.int32),
            pltpu.VMEM((4, CHUNK, OUT_DIM), jnp.float32),
            pltpu.SemaphoreType.DMA,
            pltpu.SemaphoreType.DMA,
            pltpu.SemaphoreType.DMA,
            pltpu.SemaphoreType.DMA,
            pltpu.SemaphoreType.DMA,
            pltpu.SemaphoreType.DMA,
            pltpu.SemaphoreType.DMA,
            pltpu.SemaphoreType.DMA,
            pltpu.SemaphoreType.DMA,
        ],
        compiler_params=pltpu.CompilerParams(use_tc_tiling_on_sc=False),
    )


def _norms(d):
    od = d[0, 0] + d[1, 0]
    idg = d[0, 1] + d[1, 1]
    ns = jnp.where(od > 0, lax.rsqrt(jnp.maximum(od, 1.0)), 0.0)
    nd = jnp.where(idg > 0, lax.rsqrt(jnp.maximum(idg, 1.0)), 0.0)
    return ns[:N].reshape(N, 1), nd[:N].reshape(N, 1)


def _mm1_body(h_ref, w1_ref, u_ref):
    u_ref[...] = jnp.dot(h_ref[...], w1_ref[...],
                         preferred_element_type=jnp.float32)


_tc_mm1 = pl.pallas_call(
    _mm1_body,
    out_shape=jax.ShapeDtypeStruct((N, OUT_DIM), jnp.float32),
)


def _scale_body(u_ref, deg_ref, x1_ref):
    ns, _ = _norms(deg_ref[...])
    x1_ref[:N] = u_ref[...] * ns
    x1_ref[N:] = jnp.zeros((N_PAD - N, OUT_DIM), jnp.float32)


_tc_scale = pl.pallas_call(
    _scale_body,
    out_shape=jax.ShapeDtypeStruct((N_PAD, OUT_DIM), jnp.float32),
)


def _z_body(p_ref, deg_ref, b1_ref, z_ref, zn_ref):
    ns, nd = _norms(deg_ref[...])
    p = p_ref[...]
    s = (p[0, :N] + p[1, :N]) * nd + b1_ref[...]
    z = jax.nn.sigmoid(s)
    z_ref[...] = z
    zn_ref[:N] = z * ns
    zn_ref[N:] = jnp.zeros((N_PAD - N, OUT_DIM), jnp.float32)


_tc_z = pl.pallas_call(
    _z_body,
    out_shape=(
        jax.ShapeDtypeStruct((N, OUT_DIM), jnp.float32),
        jax.ShapeDtypeStruct((N_PAD, OUT_DIM), jnp.float32),
    ),
)


def _xhat_body(q_ref, deg_ref, w2_ref, b2_ref, xh_ref):
    _, nd = _norms(deg_ref[...])
    q = q_ref[...]
    a = (q[0, :N] + q[1, :N]) * nd
    s = jnp.dot(a, w2_ref[...], preferred_element_type=jnp.float32) + b2_ref[...]
    xh_ref[...] = jax.nn.sigmoid(s)


_tc_xhat = pl.pallas_call(
    _xhat_body,
    out_shape=jax.ShapeDtypeStruct((N, IN_DIM), jnp.float32),
)


BR = 400
_GR = pl.cdiv(N, BR)


def _struct_body(zr_ref, zc_ref, out_ref):
    s = lax.dot_general(zr_ref[...], zc_ref[...],
                        (((1,), (1,)), ((), ())),
                        preferred_element_type=jnp.float32)
    out_ref[...] = jax.nn.sigmoid(s)


_tc_struct = pl.pallas_call(
    _struct_body,
    grid=(_GR,),
    in_specs=[
        pl.BlockSpec((BR, OUT_DIM), lambda i: (i, 0)),
        pl.BlockSpec((N, OUT_DIM), lambda i: (0, 0)),
    ],
    out_specs=pl.BlockSpec((BR, N), lambda i: (i, 0)),
    out_shape=jax.ShapeDtypeStruct((N, N), jnp.float32),
)


def kernel(h, edge_index, W1, b1, W2, b2):
    pad = jnp.full((EPC * CHUNK - E,), N, jnp.int32)
    src2d = jnp.concatenate([edge_index[0], pad]).reshape(EPC, CHUNK)
    dst2d = jnp.concatenate([edge_index[1], pad]).reshape(EPC, CHUNK)
    zeros_1d = jnp.zeros((ROWS_PER_TILE,), jnp.float32)
    zeros_row = jnp.zeros((ROWS_PER_TILE, OUT_DIM), jnp.float32)

    degs = _sc_degrees_kernel()(src2d, dst2d, zeros_1d)

    u = _tc_mm1(h, W1)
    x1 = _tc_scale(u, degs)
    p = _sc_segsum_kernel()(x1, src2d, dst2d, zeros_row)
    z, zn = _tc_z(p, degs, b1[None, :])
    q = _sc_segsum_kernel()(zn, src2d, dst2d, zeros_row)
    x_hat = _tc_xhat(q, degs, W2, b2[None, :])
    struct = _tc_struct(z, z)
    return (struct, x_hat)


# speedup vs baseline: 1.9410x; 1.0533x over previous
import functools

import jax
import jax.numpy as jnp
from jax import lax
from jax.experimental import pallas as pl
from jax.experimental.pallas import tpu as pltpu
from jax.experimental.pallas import tpu_sc as plsc

N = 10000
E = 160000
IN_DIM = 128
OUT_DIM = 64

NC = 2
NS = 16
CHUNK = 128
NCHUNKS = E // CHUNK
CPT0 = 40
N_PAD = 10240
ROWS_PER_TILE = N_PAD // NS
STAGE = 320


def _tile_range():
    cid = lax.axis_index("c")
    sid = lax.axis_index("s")
    wid = sid * NC + cid
    nch = (NCHUNKS // 32) + jnp.where(wid >= 30, 1, 0)
    base = (NCHUNKS // 32) * wid + jnp.maximum(wid - 30, 0)
    return cid, sid, base, nch


def _deg_body(edges, zeros_hbm, out_hbm, acc_o, acc_i, stage,
              idx_s, idx_d, ones_v, semi, sem):
    cid, sid, base, nch = _tile_range()
    row0 = sid * ROWS_PER_TILE

    cp_s = pltpu.async_copy(edges.at[0, pl.ds(base, CPT0)], idx_s, semi)
    cp_d = pltpu.async_copy(edges.at[1, pl.ds(base, CPT0)], idx_d, semi)

    for j in range(CHUNK // 16):
        ones_v[pl.ds(j * 16, 16)] = jnp.ones((16,), jnp.float32)

    pltpu.sync_copy(zeros_hbm, stage)
    pltpu.sync_copy(stage, acc_o.at[pl.ds(row0, ROWS_PER_TILE)])
    pltpu.sync_copy(stage, acc_i.at[pl.ds(row0, ROWS_PER_TILE)])
    plsc.subcore_barrier()
    cp_s.wait()
    cp_d.wait()

    def fire(c, carry):
        pltpu.async_copy(ones_v, acc_o.at[idx_s.at[c]], sem, add=True)
        pltpu.async_copy(ones_v, acc_i.at[idx_d.at[c]], sem, add=True)
        return carry

    lax.fori_loop(0, nch, fire, 0)

    def drain(c, carry):
        pltpu.make_async_copy(ones_v, acc_o.at[idx_s.at[c]], sem).wait()
        pltpu.make_async_copy(ones_v, acc_i.at[idx_d.at[c]], sem).wait()
        return carry

    lax.fori_loop(0, nch, drain, 0)
    plsc.subcore_barrier()

    pltpu.sync_copy(acc_o.at[pl.ds(row0, ROWS_PER_TILE)], stage)
    pltpu.sync_copy(stage, out_hbm.at[cid, 0, pl.ds(row0, ROWS_PER_TILE)])
    pltpu.sync_copy(acc_i.at[pl.ds(row0, ROWS_PER_TILE)], stage)
    pltpu.sync_copy(stage, out_hbm.at[cid, 1, pl.ds(row0, ROWS_PER_TILE)])


@functools.cache
def _sc_degrees_kernel():
    return pl.kernel(
        _deg_body,
        out_type=jax.ShapeDtypeStruct((NC, 2, N_PAD), jnp.float32),
        mesh=plsc.VectorSubcoreMesh(core_axis_name="c", subcore_axis_name="s",
                                    num_cores=NC, num_subcores=NS),
        scratch_types=[
            pltpu.VMEM_SHARED((N_PAD,), jnp.float32),
            pltpu.VMEM_SHARED((N_PAD,), jnp.float32),
            pltpu.VMEM((ROWS_PER_TILE,), jnp.float32),
            pltpu.VMEM((CPT0, CHUNK), jnp.int32),
            pltpu.VMEM((CPT0, CHUNK), jnp.int32),
            pltpu.VMEM((CHUNK,), jnp.float32),
            pltpu.SemaphoreType.DMA,
            pltpu.SemaphoreType.DMA,
        ],
        compiler_params=pltpu.CompilerParams(use_tc_tiling_on_sc=False),
    )


def _segsum_body(x_hbm, edges, zeros_hbm, out_hbm, acc, xsp,
                 idx_s, idx_d, rows, semi,
                 semg0, semg1, semg2, semg3, sems0, sems1, sems2, sems3):
    cid, sid, base, nch = _tile_range()
    row0 = sid * ROWS_PER_TILE
    semg = (semg0, semg1, semg2, semg3)
    sems = (sems0, sems1, sems2, sems3)

    cp_s = pltpu.async_copy(edges.at[0, pl.ds(base, CPT0)], idx_s, semi)
    cp_d = pltpu.async_copy(edges.at[1, pl.ds(base, CPT0)], idx_d, semi)

    pltpu.sync_copy(x_hbm.at[pl.ds(row0, ROWS_PER_TILE)],
                    xsp.at[pl.ds(row0, ROWS_PER_TILE)])
    pltpu.sync_copy(zeros_hbm, acc.at[pl.ds(row0, ROWS_PER_TILE)])
    plsc.subcore_barrier()
    cp_s.wait()
    cp_d.wait()

    def gathers(s, j):
        pltpu.async_copy(xsp.at[idx_s.at[s]], rows.at[j], semg[j])

    def wait_gathers(s, j):
        pltpu.make_async_copy(xsp.at[idx_s.at[s]], rows.at[j],
                              semg[j]).wait()

    def scatters(s, j):
        pltpu.async_copy(rows.at[j], acc.at[idx_d.at[s]], sems[j], add=True)

    def wait_scatters(s, j):
        pltpu.make_async_copy(rows.at[j], acc.at[idx_d.at[s]],
                              sems[j]).wait()

    gathers(0, 0)
    gathers(1, 1)

    def step(s, j):
        wait_gathers(s, j)
        scatters(s, j)
        j2 = (j + 2) % 4
        s2 = s + 2

        @pl.when(s2 < nch)
        def _():
            @pl.when(s >= 2)
            def _():
                wait_scatters(s - 2, j2)
            gathers(s2, j2)

    def body(t, carry):
        for j in range(4):
            step(t * 4 + j, j)
        return carry

    lax.fori_loop(0, nch // 4, body, 0)
    tail = (nch // 4) * 4
    for j in range(4):
        @pl.when(tail + j < nch)
        def _():
            step(tail + j, j)
    for j in range(4):
        wait_scatters(0, j)
    plsc.subcore_barrier()

    pltpu.sync_copy(acc.at[pl.ds(row0, ROWS_PER_TILE)],
                    out_hbm.at[cid, pl.ds(row0, ROWS_PER_TILE)])


@functools.cache
def _sc_segsum_kernel():
    return pl.kernel(
        _segsum_body,
        out_type=jax.ShapeDtypeStruct((NC, N_PAD, OUT_DIM), jnp.float32),
        mesh=plsc.VectorSubcoreMesh(core_axis_name="c", subcore_axis_name="s",
                                    num_cores=NC, num_subcores=NS),
        scratch_types=[
            pltpu.VMEM_SHARED((N_PAD, OUT_DIM), jnp.float32),
            pltpu.VMEM_SHARED((N_PAD, OUT_DIM), jnp.float32),
            pltpu.VMEM((CPT0, CHUNK), jnp.int32),
            pltpu.VMEM((CPT0, CHUNK), jnp.int32),
            pltpu.VMEM((4, CHUNK, OUT_DIM), jnp.float32),
            pltpu.SemaphoreType.DMA,
            pltpu.SemaphoreType.DMA,
            pltpu.SemaphoreType.DMA,
            pltpu.SemaphoreType.DMA,
            pltpu.SemaphoreType.DMA,
            pltpu.SemaphoreType.DMA,
            pltpu.SemaphoreType.DMA,
            pltpu.SemaphoreType.DMA,
            pltpu.SemaphoreType.DMA,
        ],
        compiler_params=pltpu.CompilerParams(use_tc_tiling_on_sc=False),
    )


def _norms(d):
    od = d[0, 0] + d[1, 0]
    idg = d[0, 1] + d[1, 1]
    ns = jnp.where(od > 0, lax.rsqrt(jnp.maximum(od, 1.0)), 0.0)
    nd = jnp.where(idg > 0, lax.rsqrt(jnp.maximum(idg, 1.0)), 0.0)
    return ns[:N].reshape(N, 1), nd[:N].reshape(N, 1)


def _mm1_body(h_ref, w1_ref, u_ref):
    u_ref[...] = jnp.dot(h_ref[...], w1_ref[...],
                         preferred_element_type=jnp.float32)


_tc_mm1 = pl.pallas_call(
    _mm1_body,
    out_shape=jax.ShapeDtypeStruct((N, OUT_DIM), jnp.float32),
)


def _scale_body(u_ref, deg_ref, x1_ref):
    ns, _ = _norms(deg_ref[...])
    x1_ref[:N] = u_ref[...] * ns
    x1_ref[N:] = jnp.zeros((N_PAD - N, OUT_DIM), jnp.float32)


_tc_scale = pl.pallas_call(
    _scale_body,
    out_shape=jax.ShapeDtypeStruct((N_PAD, OUT_DIM), jnp.float32),
)


def _z_body(p_ref, deg_ref, b1_ref, z_ref, zn_ref):
    ns, nd = _norms(deg_ref[...])
    p = p_ref[...]
    s = (p[0, :N] + p[1, :N]) * nd + b1_ref[...]
    z = jax.nn.sigmoid(s)
    z_ref[...] = z
    zn_ref[:N] = z * ns
    zn_ref[N:] = jnp.zeros((N_PAD - N, OUT_DIM), jnp.float32)


_tc_z = pl.pallas_call(
    _z_body,
    out_shape=(
        jax.ShapeDtypeStruct((N, OUT_DIM), jnp.float32),
        jax.ShapeDtypeStruct((N_PAD, OUT_DIM), jnp.float32),
    ),
)


def _xhat_body(q_ref, deg_ref, w2_ref, b2_ref, xh_ref):
    _, nd = _norms(deg_ref[...])
    q = q_ref[...]
    a = (q[0, :N] + q[1, :N]) * nd
    s = jnp.dot(a, w2_ref[...], preferred_element_type=jnp.float32) + b2_ref[...]
    xh_ref[...] = jax.nn.sigmoid(s)


_tc_xhat = pl.pallas_call(
    _xhat_body,
    out_shape=jax.ShapeDtypeStruct((N, IN_DIM), jnp.float32),
)


BR = 400
_GR = pl.cdiv(N, BR)


def _struct_body(zr_ref, zc_ref, out_ref):
    s = lax.dot_general(zr_ref[...], zc_ref[...],
                        (((1,), (1,)), ((), ())),
                        preferred_element_type=jnp.float32)
    out_ref[...] = jax.nn.sigmoid(s)


_tc_struct = pl.pallas_call(
    _struct_body,
    grid=(_GR,),
    in_specs=[
        pl.BlockSpec((BR, OUT_DIM), lambda i: (i, 0)),
        pl.BlockSpec((N, OUT_DIM), lambda i: (0, 0)),
    ],
    out_specs=pl.BlockSpec((BR, N), lambda i: (i, 0)),
    out_shape=jax.ShapeDtypeStruct((N, N), jnp.float32),
)


def kernel(h, edge_index, W1, b1, W2, b2):
    edge3 = edge_index.reshape(2, NCHUNKS, CHUNK)
    zeros_1d = jnp.zeros((ROWS_PER_TILE,), jnp.float32)
    zeros_row = jnp.zeros((ROWS_PER_TILE, OUT_DIM), jnp.float32)

    degs = _sc_degrees_kernel()(edge3, zeros_1d)

    u = _tc_mm1(h, W1)
    x1 = _tc_scale(u, degs)
    p = _sc_segsum_kernel()(x1, edge3, zeros_row)
    z, zn = _tc_z(p, degs, b1[None, :])
    q = _sc_segsum_kernel()(zn, edge3, zeros_row)
    x_hat = _tc_xhat(q, degs, W2, b2[None, :])
    struct = _tc_struct(z, z)
    return (struct, x_hat)
